# baseline (speedup 1.0000x reference)
import functools

import jax
import jax.numpy as jnp
from jax import lax
from jax.experimental import pallas as pl
from jax.experimental.pallas import tpu as pltpu
from jax.experimental.pallas import tpu_sc as plsc

N_EMBD = 4096
N_EXPERTS = 64
TOP_K = 8

_TOKEN_BLOCK = 1024
_LANES = 16
_NUM_WORKERS = 32
_CHUNKS = 2


def _matmul_block(x_ref, wt_ref, l_out_ref, lt_out_ref):
    logits = jax.lax.dot_general(
        x_ref[...], wt_ref[...],
        dimension_numbers=(((1,), (0,)), ((), ())),
        preferred_element_type=jnp.float32,
    )
    l_out_ref[...] = logits
    lt_out_ref[...] = logits.T


def _gate_logits(xf, wt, interpret=False):
    n_tok = xf.shape[0]
    c = xf.shape[1]
    return pl.pallas_call(
        _matmul_block,
        grid=(n_tok // _TOKEN_BLOCK,),
        in_specs=[
            pl.BlockSpec((_TOKEN_BLOCK, c), lambda i: (i, 0)),
            pl.BlockSpec((c, N_EXPERTS), lambda i: (0, 0)),
        ],
        out_specs=[
            pl.BlockSpec((_TOKEN_BLOCK, N_EXPERTS), lambda i: (i, 0)),
            pl.BlockSpec((N_EXPERTS, _TOKEN_BLOCK), lambda i: (0, i)),
        ],
        out_shape=[
            jax.ShapeDtypeStruct((n_tok, N_EXPERTS), jnp.float32),
            jax.ShapeDtypeStruct((N_EXPERTS, n_tok), jnp.float32),
        ],
        interpret=interpret,
    )(xf, wt)


def _sc_router_body(n_per_w, lt_hbm, w_hbm, i_hbm, lt_vmem, w_vmem, i_vmem):
    nc = 2
    wid = lax.axis_index("s") * nc + lax.axis_index("c")
    base = wid * n_per_w
    pltpu.sync_copy(lt_hbm.at[:, pl.ds(base, n_per_w)], lt_vmem)

    n_groups = n_per_w // _LANES

    def group_body(g, carry):
        off = g * _LANES
        r = [jnp.full((_LANES,), -jnp.inf, jnp.float32) for _ in range(TOP_K)]
        ri = [jnp.zeros((_LANES,), jnp.int32) for _ in range(TOP_K)]
        for e in range(N_EXPERTS):
            v = lt_vmem[e, pl.ds(off, _LANES)]
            vi = jnp.full((_LANES,), e, jnp.int32)
            for k in range(TOP_K):
                m = v > r[k]
                r[k], v = jnp.where(m, v, r[k]), jnp.where(m, r[k], v)
                ri[k], vi = jnp.where(m, vi, ri[k]), jnp.where(m, ri[k], vi)
        mx = r[0]
        es = [jnp.exp(rk - mx) for rk in r]
        s = es[0]
        for k in range(1, TOP_K):
            s = s + es[k]
        inv = jnp.float32(1.0) / s
        for k in range(TOP_K):
            w_vmem[k, pl.ds(off, _LANES)] = es[k] * inv
            i_vmem[k, pl.ds(off, _LANES)] = ri[k]
        return carry

    lax.fori_loop(0, n_groups, group_body, 0)
    pltpu.sync_copy(w_vmem, w_hbm.at[:, pl.ds(base, n_per_w)])
    pltpu.sync_copy(i_vmem, i_hbm.at[:, pl.ds(base, n_per_w)])


def _sc_router(logits_t, interpret=False):
    n_tok = logits_t.shape[1]
    n_per_w = n_tok // _NUM_WORKERS
    mesh = plsc.VectorSubcoreMesh(core_axis_name="c", subcore_axis_name="s",
                                  num_cores=2, num_subcores=16)
    return pl.kernel(
        functools.partial(_sc_router_body, n_per_w),
        out_type=[
            jax.ShapeDtypeStruct((TOP_K, n_tok), jnp.float32),
            jax.ShapeDtypeStruct((TOP_K, n_tok), jnp.int32),
        ],
        mesh=mesh,
        scratch_types=[
            pltpu.VMEM((N_EXPERTS, n_per_w), jnp.float32),
            pltpu.VMEM((TOP_K, n_per_w), jnp.float32),
            pltpu.VMEM((TOP_K, n_per_w), jnp.int32),
        ],
        interpret=interpret,
    )(logits_t)


@functools.partial(jax.jit, static_argnames=("interpret",))
def kernel(x, W, interpret=False):
    b, t, c = x.shape
    n_tok = b * t
    xf = x.reshape(n_tok, c)
    wt = W.T

    n_chunk = n_tok // _CHUNKS
    logit_parts = []
    wt_parts = []
    it_parts = []
    for ci in range(_CHUNKS):
        xc = jax.lax.slice_in_dim(xf, ci * n_chunk, (ci + 1) * n_chunk, axis=0)
        logits_c, logits_t_c = _gate_logits(xc, wt, interpret=interpret)
        w_t_c, i_t_c = _sc_router(logits_t_c, interpret=interpret)
        logit_parts.append(logits_c)
        wt_parts.append(w_t_c)
        it_parts.append(i_t_c)
    logits = jnp.concatenate(logit_parts, axis=0)
    weights = jnp.concatenate(wt_parts, axis=1).T
    indices = jnp.concatenate(it_parts, axis=1).T

    return (weights.reshape(b, t, TOP_K),
            indices.reshape(b, t, TOP_K),
            logits.reshape(b, t, N_EXPERTS))


# speedup vs baseline: 2.0173x; 2.0173x over previous
import functools

import jax
import jax.numpy as jnp
from jax import lax
from jax.experimental import pallas as pl
from jax.experimental.pallas import tpu as pltpu
from jax.experimental.pallas import tpu_sc as plsc

N_EMBD = 4096
N_EXPERTS = 64
TOP_K = 8

_TOKEN_BLOCK = 1024
_LANES = 16
_NUM_WORKERS = 32
_CHUNKS = 2


def _matmul_block(x_ref, wt_ref, l_out_ref, lt_out_ref):
    logits = jax.lax.dot_general(
        x_ref[...], wt_ref[...],
        dimension_numbers=(((1,), (0,)), ((), ())),
        preferred_element_type=jnp.float32,
    )
    l_out_ref[...] = logits
    lt_out_ref[...] = logits.T


def _gate_logits(xf, wt, ci, n_chunk, interpret=False):
    c = xf.shape[1]
    blk0 = ci * (n_chunk // _TOKEN_BLOCK)
    return pl.pallas_call(
        _matmul_block,
        grid=(n_chunk // _TOKEN_BLOCK,),
        in_specs=[
            pl.BlockSpec((_TOKEN_BLOCK, c), lambda i: (blk0 + i, 0)),
            pl.BlockSpec((c, N_EXPERTS), lambda i: (0, 0)),
        ],
        out_specs=[
            pl.BlockSpec((_TOKEN_BLOCK, N_EXPERTS), lambda i: (i, 0)),
            pl.BlockSpec((N_EXPERTS, _TOKEN_BLOCK), lambda i: (0, i)),
        ],
        out_shape=[
            jax.ShapeDtypeStruct((n_chunk, N_EXPERTS), jnp.float32),
            jax.ShapeDtypeStruct((N_EXPERTS, n_chunk), jnp.float32),
        ],
        interpret=interpret,
    )(xf, wt)


def _sc_router_body(n_per_w, lt_hbm, w_hbm, i_hbm, lt_vmem, w_vmem, i_vmem):
    nc = 2
    wid = lax.axis_index("s") * nc + lax.axis_index("c")
    base = wid * n_per_w
    pltpu.sync_copy(lt_hbm.at[:, pl.ds(base, n_per_w)], lt_vmem)

    n_groups = n_per_w // _LANES

    def group_body(g, carry):
        off = g * _LANES
        r = [jnp.full((_LANES,), -jnp.inf, jnp.float32) for _ in range(TOP_K)]
        ri = [jnp.zeros((_LANES,), jnp.int32) for _ in range(TOP_K)]
        for e in range(N_EXPERTS):
            v = lt_vmem[e, pl.ds(off, _LANES)]
            vi = jnp.full((_LANES,), e, jnp.int32)
            for k in range(TOP_K):
                m = v > r[k]
                r[k], v = jnp.where(m, v, r[k]), jnp.where(m, r[k], v)
                ri[k], vi = jnp.where(m, vi, ri[k]), jnp.where(m, ri[k], vi)
        mx = r[0]
        es = [jnp.exp(rk - mx) for rk in r]
        s = es[0]
        for k in range(1, TOP_K):
            s = s + es[k]
        inv = jnp.float32(1.0) / s
        for k in range(TOP_K):
            w_vmem[k, pl.ds(off, _LANES)] = es[k] * inv
            i_vmem[k, pl.ds(off, _LANES)] = ri[k]
        return carry

    lax.fori_loop(0, n_groups, group_body, 0)
    pltpu.sync_copy(w_vmem, w_hbm.at[:, pl.ds(base, n_per_w)])
    pltpu.sync_copy(i_vmem, i_hbm.at[:, pl.ds(base, n_per_w)])


def _sc_router(logits_t, interpret=False):
    n_tok = logits_t.shape[1]
    n_per_w = n_tok // _NUM_WORKERS
    mesh = plsc.VectorSubcoreMesh(core_axis_name="c", subcore_axis_name="s",
                                  num_cores=2, num_subcores=16)
    return pl.kernel(
        functools.partial(_sc_router_body, n_per_w),
        out_type=[
            jax.ShapeDtypeStruct((TOP_K, n_tok), jnp.float32),
            jax.ShapeDtypeStruct((TOP_K, n_tok), jnp.int32),
        ],
        mesh=mesh,
        scratch_types=[
            pltpu.VMEM((N_EXPERTS, n_per_w), jnp.float32),
            pltpu.VMEM((TOP_K, n_per_w), jnp.float32),
            pltpu.VMEM((TOP_K, n_per_w), jnp.int32),
        ],
        interpret=interpret,
    )(logits_t)


@functools.partial(jax.jit, static_argnames=("interpret",))
def kernel(x, W, interpret=False):
    b, t, c = x.shape
    n_tok = b * t
    xf = x.reshape(n_tok, c)
    wt = W.T

    n_chunk = n_tok // _CHUNKS
    logit_parts = []
    wt_parts = []
    it_parts = []
    for ci in range(_CHUNKS):
        logits_c, logits_t_c = _gate_logits(xf, wt, ci, n_chunk,
                                             interpret=interpret)
        w_t_c, i_t_c = _sc_router(logits_t_c, interpret=interpret)
        logit_parts.append(logits_c)
        wt_parts.append(w_t_c)
        it_parts.append(i_t_c)
    logits = jnp.concatenate(logit_parts, axis=0)
    weights = jnp.concatenate(wt_parts, axis=1).T
    indices = jnp.concatenate(it_parts, axis=1).T

    return (weights.reshape(b, t, TOP_K),
            indices.reshape(b, t, TOP_K),
            logits.reshape(b, t, N_EXPERTS))


# speedup vs baseline: 2.3917x; 1.1856x over previous
import functools

import jax
import jax.numpy as jnp
from jax.experimental import pallas as pl
from jax.experimental.pallas import tpu as pltpu

N_EMBD = 4096
N_EXPERTS = 64
TOP_K = 8

_TOKEN_BLOCK = 1024


def _topk_softmax(logits):
    tb = logits.shape[0]
    lane_f = jax.lax.broadcasted_iota(
        jnp.int32, (tb, N_EXPERTS), 1).astype(jnp.float32)
    big = jnp.float32(N_EXPERTS)
    cur = logits
    vals = []
    idxs_f = []
    for _ in range(TOP_K):
        m = jnp.max(cur, axis=1, keepdims=True)
        sel = jnp.min(jnp.where(cur == m, lane_f, big), axis=1, keepdims=True)
        vals.append(m)
        idxs_f.append(sel)
        cur = jnp.where(lane_f == sel, -jnp.inf, cur)
    top_vals = jnp.concatenate(vals, axis=1)
    top_idxs = jnp.concatenate(idxs_f, axis=1).astype(jnp.int32)

    e = jnp.exp(top_vals - top_vals[:, 0:1])
    return e / jnp.sum(e, axis=1, keepdims=True), top_idxs


def _router_block(x_ref, wt_ref, w_out_ref, i_out_ref, l_out_ref,
                  prev_ref, *, n_blocks):
    i = pl.program_id(0)
    logits = jax.lax.dot_general(
        x_ref[...], wt_ref[...],
        dimension_numbers=(((1,), (0,)), ((), ())),
        preferred_element_type=jnp.float32,
    )
    l_out_ref[...] = logits

    @pl.when(i > 0)
    def _prev():
        w, ix = _topk_softmax(prev_ref[...])
        base = (i - 1) * _TOKEN_BLOCK
        w_out_ref[pl.ds(base, _TOKEN_BLOCK), :] = w
        i_out_ref[pl.ds(base, _TOKEN_BLOCK), :] = ix

    @pl.when(i < n_blocks - 1)
    def _stash():
        prev_ref[...] = logits

    @pl.when(i == n_blocks - 1)
    def _last():
        w, ix = _topk_softmax(logits)
        base = i * _TOKEN_BLOCK
        w_out_ref[pl.ds(base, _TOKEN_BLOCK), :] = w
        i_out_ref[pl.ds(base, _TOKEN_BLOCK), :] = ix


@functools.partial(jax.jit, static_argnames=("interpret",))
def kernel(x, W, interpret=False):
    b, t, c = x.shape
    n_tok = b * t
    xf = x.reshape(n_tok, c)
    wt = W.T
    n_blocks = n_tok // _TOKEN_BLOCK

    weights, indices, logits = pl.pallas_call(
        functools.partial(_router_block, n_blocks=n_blocks),
        grid=(n_blocks,),
        in_specs=[
            pl.BlockSpec((_TOKEN_BLOCK, c), lambda i: (i, 0)),
            pl.BlockSpec((c, N_EXPERTS), lambda i: (0, 0)),
        ],
        out_specs=[
            pl.BlockSpec((n_tok, TOP_K), lambda i: (0, 0)),
            pl.BlockSpec((n_tok, TOP_K), lambda i: (0, 0)),
            pl.BlockSpec((_TOKEN_BLOCK, N_EXPERTS), lambda i: (i, 0)),
        ],
        out_shape=[
            jax.ShapeDtypeStruct((n_tok, TOP_K), jnp.float32),
            jax.ShapeDtypeStruct((n_tok, TOP_K), jnp.int32),
            jax.ShapeDtypeStruct((n_tok, N_EXPERTS), jnp.float32),
        ],
        scratch_shapes=[pltpu.VMEM((_TOKEN_BLOCK, N_EXPERTS), jnp.float32)],
        interpret=interpret,
    )(xf, wt)

    return (weights.reshape(b, t, TOP_K),
            indices.reshape(b, t, TOP_K),
            logits.reshape(b, t, N_EXPERTS))
